# Initial kernel scaffold; baseline (speedup 1.0000x reference)
import jax
import jax.numpy as jnp
from jax.experimental import pallas as pl


def kernel(user_embed, item_embed, adj_rows, adj_cols, adj_vals):
    raise NotImplementedError("write your pallas kernel here")


# speedup vs baseline: 3.7356x; 3.7356x over previous
import jax
import jax.numpy as jnp
from jax import lax
from jax.experimental import pallas as pl
from jax.experimental.pallas import tpu as pltpu
from jax.experimental.pallas import tpu_sc as plsc

N_NODES_K = 10000
N_PAD = 10240
D_K = 128
NNZ_K = 320000

NUM_CORES = 2
NUM_SUBCORES = 16
NUM_WORKERS = NUM_CORES * NUM_SUBCORES
EDGES_PER_WORKER = NNZ_K // NUM_WORKERS
BLK = 80
NBLK = EDGES_PER_WORKER // BLK
ROWS_PER_TILE = N_PAD // NUM_SUBCORES


def _hop_body(t_hbm, cols_hbm, rows_hbm, vals_hbm, out_hbm,
              cols_b, rows_b, vals_b, gbuf, acc, gsem):
    c = lax.axis_index("c")
    s = lax.axis_index("s")
    w = s * NUM_CORES + c
    ebase = w * EDGES_PER_WORKER

    def zero_chunk(i, _):
        gbuf[i // 8, pl.ds((i % 8) * 16, 16)] = jnp.zeros((16,), jnp.float32)
        return 0

    lax.fori_loop(0, BLK * 8, zero_chunk, 0)
    for k in range(ROWS_PER_TILE // BLK):
        pltpu.sync_copy(gbuf, acc.at[pl.ds(s * ROWS_PER_TILE + k * BLK, BLK)])

    plsc.subcore_barrier()

    def block_body(b, _):
        off = ebase + b * BLK
        pltpu.sync_copy(cols_hbm.at[pl.ds(off, BLK)], cols_b)
        pltpu.sync_copy(rows_hbm.at[pl.ds(off, BLK)], rows_b)
        pltpu.sync_copy(vals_hbm.at[pl.ds(off, BLK)], vals_b)
        pltpu.async_copy(t_hbm.at[cols_b], gbuf, gsem).wait()

        def scale_group(eb, _):
            val16 = vals_b[pl.ds(eb * 16, 16)]
            for l in range(16):
                v = val16[l]
                e = eb * 16 + l
                for j in range(8):
                    sl = (e, pl.ds(j * 16, 16))
                    gbuf[sl] = gbuf[sl] * v
            return 0

        lax.fori_loop(0, BLK // 16, scale_group, 0)
        pltpu.sync_copy(gbuf, acc.at[rows_b], add=True)
        return 0

    lax.fori_loop(0, NBLK, block_body, 0)

    plsc.subcore_barrier()

    for k in range(ROWS_PER_TILE // BLK):
        off = s * ROWS_PER_TILE + k * BLK
        pltpu.sync_copy(acc.at[pl.ds(off, BLK)], out_hbm.at[c, pl.ds(off, BLK)])


@jax.jit
def _hop(t, cols, rows, vals):
    mesh = plsc.VectorSubcoreMesh(core_axis_name="c", subcore_axis_name="s")
    f = pl.kernel(
        _hop_body,
        mesh=mesh,
        out_type=jax.ShapeDtypeStruct((NUM_CORES, N_PAD, D_K), jnp.float32),
        scratch_types=[
            pltpu.VMEM((BLK,), jnp.int32),
            pltpu.VMEM((BLK,), jnp.int32),
            pltpu.VMEM((BLK,), jnp.float32),
            pltpu.VMEM((BLK, D_K), jnp.float32),
            pltpu.VMEM_SHARED((N_PAD, D_K), jnp.float32),
            pltpu.SemaphoreType.DMA,
        ],
    )
    return f(t, cols, rows, vals)


def _add_body(a_ref, b_ref, o_ref):
    o_ref[...] = a_ref[...] + b_ref[...]


@jax.jit
def _combine(p):
    spec = pl.BlockSpec((1024, D_K), lambda i: (i, 0))
    return pl.pallas_call(
        _add_body,
        grid=(N_PAD // 1024,),
        in_specs=[spec, spec],
        out_specs=spec,
        out_shape=jax.ShapeDtypeStruct((N_PAD, D_K), jnp.float32),
    )(p[0], p[1])


def kernel(user_embed, item_embed, adj_rows, adj_cols, adj_vals):
    t0 = jnp.concatenate(
        [user_embed, item_embed,
         jnp.zeros((N_PAD - N_NODES_K, D_K), jnp.float32)], axis=0)

    embs = [t0]
    t = t0
    for _ in range(3):
        p = _hop(t, adj_cols, adj_rows, adj_vals)
        t = _combine(p)
        embs.append(t)
    embs = jnp.stack(embs, axis=1)
    n_users = user_embed.shape[0]
    return embs[:n_users], embs[n_users:N_NODES_K]


# speedup vs baseline: 7.6468x; 2.0470x over previous
import jax
import jax.numpy as jnp
from jax import lax
from jax.experimental import pallas as pl
from jax.experimental.pallas import tpu as pltpu
from jax.experimental.pallas import tpu_sc as plsc

N_NODES_K = 10000
N_PAD = 10240
D_K = 128
NNZ_K = 320000

NUM_CORES = 2
NUM_SUBCORES = 16
NUM_WORKERS = NUM_CORES * NUM_SUBCORES
EDGES_PER_WORKER = NNZ_K // NUM_WORKERS
BLK = 80
NBLK = EDGES_PER_WORKER // BLK
ROWS_PER_TILE = N_PAD // NUM_SUBCORES


def _hop_body(t_hbm, cols_hbm, rows_hbm, vals_hbm, out_hbm,
              cols0, cols1, rows0, rows1, vals0, vals1, gbuf0, gbuf1,
              acc, semg0, semg1, semm0, semm1):
    c = lax.axis_index("c")
    s = lax.axis_index("s")
    w = s * NUM_CORES + c
    ebase = w * EDGES_PER_WORKER

    cols = (cols0, cols1)
    rows = (rows0, rows1)
    vals = (vals0, vals1)
    gbuf = (gbuf0, gbuf1)
    semg = (semg0, semg1)
    semm = (semm0, semm1)

    def zero_chunk(i, _):
        gbuf0[i // 8, pl.ds((i % 8) * 16, 16)] = jnp.zeros((16,), jnp.float32)
        return 0

    lax.fori_loop(0, BLK * 8, zero_chunk, 0)
    for k in range(ROWS_PER_TILE // BLK):
        pltpu.sync_copy(gbuf0, acc.at[pl.ds(s * ROWS_PER_TILE + k * BLK, BLK)])

    plsc.subcore_barrier()

    def meta_issue(m, x):
        off = ebase + jnp.minimum(m, NBLK - 1) * BLK
        pltpu.async_copy(cols_hbm.at[pl.ds(off, BLK)], cols[x], semm[x])
        pltpu.async_copy(rows_hbm.at[pl.ds(off, BLK)], rows[x], semm[x])
        pltpu.async_copy(vals_hbm.at[pl.ds(off, BLK)], vals[x], semm[x])

    def meta_wait(x):
        pltpu.make_async_copy(cols_hbm.at[pl.ds(0, BLK)], cols[x], semm[x]).wait()
        pltpu.make_async_copy(rows_hbm.at[pl.ds(0, BLK)], rows[x], semm[x]).wait()
        pltpu.make_async_copy(vals_hbm.at[pl.ds(0, BLK)], vals[x], semm[x]).wait()

    def process(n, x, prefetch):
        y = 1 - x
        pltpu.make_async_copy(t_hbm.at[cols[x]], gbuf[x], semg[x]).wait()
        if prefetch:
            meta_wait(y)
            pltpu.async_copy(t_hbm.at[cols[y]], gbuf[y], semg[y])

        def scale_group(eb, _):
            val16 = vals[x][pl.ds(eb * 16, 16)]
            for l in range(16):
                v = val16[l]
                e = eb * 16 + l
                for j in range(8):
                    sl = (e, pl.ds(j * 16, 16))
                    gbuf[x][sl] = gbuf[x][sl] * v
            return 0

        lax.fori_loop(0, BLK // 16, scale_group, 0)
        pltpu.sync_copy(gbuf[x], acc.at[rows[x]], add=True)
        if prefetch:
            meta_issue(n + 2, x)

    pltpu.sync_copy(cols_hbm.at[pl.ds(ebase, BLK)], cols0)
    pltpu.sync_copy(rows_hbm.at[pl.ds(ebase, BLK)], rows0)
    pltpu.sync_copy(vals_hbm.at[pl.ds(ebase, BLK)], vals0)
    pltpu.async_copy(t_hbm.at[cols0], gbuf0, semg0)
    meta_issue(1, 1)

    def pair_body(p, _):
        process(2 * p, 0, True)
        process(2 * p + 1, 1, True)
        return 0

    lax.fori_loop(0, (NBLK - 1) // 2, pair_body, 0)
    process(NBLK - 1, 0, False)
    meta_wait(1)

    plsc.subcore_barrier()

    for k in range(ROWS_PER_TILE // BLK):
        off = s * ROWS_PER_TILE + k * BLK
        pltpu.sync_copy(acc.at[pl.ds(off, BLK)], out_hbm.at[c, pl.ds(off, BLK)])


@jax.jit
def _hop(t, cols, rows, vals):
    mesh = plsc.VectorSubcoreMesh(core_axis_name="c", subcore_axis_name="s")
    f = pl.kernel(
        _hop_body,
        mesh=mesh,
        out_type=jax.ShapeDtypeStruct((NUM_CORES, N_PAD, D_K), jnp.float32),
        scratch_types=[
            pltpu.VMEM((BLK,), jnp.int32),
            pltpu.VMEM((BLK,), jnp.int32),
            pltpu.VMEM((BLK,), jnp.int32),
            pltpu.VMEM((BLK,), jnp.int32),
            pltpu.VMEM((BLK,), jnp.float32),
            pltpu.VMEM((BLK,), jnp.float32),
            pltpu.VMEM((BLK, D_K), jnp.float32),
            pltpu.VMEM((BLK, D_K), jnp.float32),
            pltpu.VMEM_SHARED((N_PAD, D_K), jnp.float32),
            pltpu.SemaphoreType.DMA,
            pltpu.SemaphoreType.DMA,
            pltpu.SemaphoreType.DMA,
            pltpu.SemaphoreType.DMA,
        ],
    )
    return f(t, cols, rows, vals)


def _add_body(a_ref, b_ref, o_ref):
    o_ref[...] = a_ref[...] + b_ref[...]


@jax.jit
def _combine(p):
    spec = pl.BlockSpec((1024, D_K), lambda i: (i, 0))
    return pl.pallas_call(
        _add_body,
        grid=(N_PAD // 1024,),
        in_specs=[spec, spec],
        out_specs=spec,
        out_shape=jax.ShapeDtypeStruct((N_PAD, D_K), jnp.float32),
    )(p[0], p[1])


def kernel(user_embed, item_embed, adj_rows, adj_cols, adj_vals):
    t0 = jnp.concatenate(
        [user_embed, item_embed,
         jnp.zeros((N_PAD - N_NODES_K, D_K), jnp.float32)], axis=0)

    embs = [t0]
    t = t0
    for _ in range(3):
        p = _hop(t, adj_cols, adj_rows, adj_vals)
        t = _combine(p)
        embs.append(t)
    embs = jnp.stack(embs, axis=1)
    n_users = user_embed.shape[0]
    return embs[:n_users], embs[n_users:N_NODES_K]
